# baseline (speedup 1.0000x reference)
import functools

import jax
import jax.numpy as jnp
from jax import lax
from jax.experimental import pallas as pl
from jax.experimental.pallas import tpu as pltpu
from jax.experimental.pallas import tpu_sc as plsc

_L = 16


def _flat_view(table):
    n, d = table.shape
    t = table.reshape(n // 128, 128, d // 8, 8)
    return t.transpose(2, 0, 3, 1).reshape(-1)


@functools.lru_cache(maxsize=None)
def _build(B, D, NUM, n_rows_a, n_rows_b):
    info = plsc.get_sparse_core_info()
    NC, NS = info.num_cores, info.num_subcores
    NW = NC * NS
    assert B % (8 * NW) == 0 and D == _L
    n_per_w = B // NW
    n_groups = n_per_w // _L
    n_flat = n_per_w * D

    mesh = plsc.VectorSubcoreMesh(core_axis_name="c", subcore_axis_name="s")

    @functools.partial(
        pl.kernel,
        mesh=mesh,
        compiler_params=pltpu.CompilerParams(
            needs_layout_passes=False, use_tc_tiling_on_sc=False),
        out_type=jax.ShapeDtypeStruct((B,), jnp.float32),
        scratch_types=[
            pltpu.VMEM((n_per_w,), jnp.int32),
            pltpu.VMEM((n_per_w,), jnp.int32),
            pltpu.VMEM((n_per_w,), jnp.int32),
            pltpu.VMEM((n_flat,), jnp.int32),
            pltpu.VMEM((n_flat,), jnp.int32),
            pltpu.VMEM((n_flat,), jnp.float32),
            pltpu.VMEM((n_flat,), jnp.float32),
            pltpu.VMEM((n_per_w,), jnp.float32),
            pltpu.SemaphoreType.DMA,
            pltpu.SemaphoreType.DMA,
            pltpu.SemaphoreType.DMA,
            pltpu.SemaphoreType.DMA,
        ],
    )
    def k(layer_hbm, aidx_hbm, bidx_hbm, a_hbm, b_hbm, out_hbm,
          layer_v, aidx_v, bidx_v, idxa_v, idxb_v, arows_v, brows_v, out_v,
          sem_a0, sem_a1, sem_b0, sem_b1):
        wid = lax.axis_index("s") * NC + lax.axis_index("c")
        base = wid * n_per_w
        sems_a = (sem_a0, sem_a1)
        sems_b = (sem_b0, sem_b1)
        n_chunks = 4
        cg = n_groups // n_chunks
        cn = n_per_w // n_chunks
        assert cg == 8 and n_chunks * cg == n_groups

        pltpu.sync_copy(layer_hbm.at[pl.ds(base, n_per_w)], layer_v)
        pltpu.sync_copy(aidx_hbm.at[pl.ds(base, n_per_w)], aidx_v)
        pltpu.sync_copy(bidx_hbm.at[pl.ds(base, n_per_w)], bidx_v)

        def strip(g, d):
            return ((g >> 3) * (_L * cn)) + (d * cn) + ((g & 7) * _L)

        def idx_body(g, _):
            off = g * _L
            sl = pl.ds(off, _L)
            ra = layer_v[sl] * NUM + aidx_v[sl]
            rb = bidx_v[sl]
            jca = ((ra >> 7) << 10) + (ra & 127)
            jcb = ((rb >> 7) << 10) + (rb & 127)
            for d in range(_L):
                ca = ((d >> 3) * (8 * n_rows_a)) + ((d & 7) << 7)
                cb = ((d >> 3) * (8 * n_rows_b)) + ((d & 7) << 7)
                idxa_v[pl.ds(strip(g, d), _L)] = jca + ca
                idxb_v[pl.ds(strip(g, d), _L)] = jcb + cb
            return 0

        def dot_body(g, _):
            acc = jnp.zeros((_L,), jnp.float32)
            for d in range(_L):
                sl = pl.ds(strip(g, d), _L)
                acc = acc + arows_v[sl] * brows_v[sl]
            out_v[pl.ds(g * _L, _L)] = acc
            return 0

        def build(c):
            lax.fori_loop(c * cg, (c + 1) * cg, idx_body, 0)

        def fire(c):
            sl = pl.ds(c * _L * cn, _L * cn)
            return [
                pltpu.async_copy(
                    a_hbm.at[idxa_v.at[sl]], arows_v.at[sl], sems_a[c % 2]),
                pltpu.async_copy(
                    b_hbm.at[idxb_v.at[sl]], brows_v.at[sl], sems_b[c % 2]),
            ]

        def drain(cps):
            for cp in cps:
                cp.wait()

        def dot(c):
            lax.fori_loop(c * cg, (c + 1) * cg, dot_body, 0)

        build(0)
        cps0 = fire(0)
        build(1)
        cps1 = fire(1)
        drain(cps0)
        dot(0)
        build(2)
        cps0 = fire(2)
        drain(cps1)
        dot(1)
        build(3)
        cps1 = fire(3)
        drain(cps0)
        dot(2)
        drain(cps1)
        dot(3)

        pltpu.sync_copy(out_v, out_hbm.at[pl.ds(base, n_per_w)])

    return k


def kernel(layerIdx, aIdx, bIdx, A_table, B_table):
    B = layerIdx.shape[0]
    NUM, D = B_table.shape
    n_rows_a = A_table.shape[0]
    assert n_rows_a % 128 == 0 and D % 8 == 0
    pad_b = (-NUM) % 128
    b_padded = jnp.pad(B_table, ((0, pad_b), (0, 0)))
    a_flat = _flat_view(A_table)
    b_flat = _flat_view(b_padded)
    k = _build(B, D, NUM, n_rows_a, NUM + pad_b)
    return k(layerIdx.astype(jnp.int32), aIdx.astype(jnp.int32),
             bIdx.astype(jnp.int32), a_flat, b_flat)


# speedup vs baseline: 1.1567x; 1.1567x over previous
import functools

import jax
import jax.numpy as jnp
from jax import lax
from jax.experimental import pallas as pl
from jax.experimental.pallas import tpu as pltpu
from jax.experimental.pallas import tpu_sc as plsc

_L = 16


def _flat_view(table):
    n, d = table.shape
    t = table.reshape(n // 128, 128, d // 8, 8)
    return t.transpose(2, 0, 3, 1).reshape(-1)


@functools.lru_cache(maxsize=None)
def _build(B, D, NUM, n_rows_a, n_rows_b):
    info = plsc.get_sparse_core_info()
    NC, NS = info.num_cores, info.num_subcores
    NW = NC * NS
    assert B % (8 * NW) == 0 and D == _L
    n_per_w = B // NW
    n_groups = n_per_w // _L
    n_flat = n_per_w * D
    b_elems = n_rows_b * D
    assert b_elems % (8 * NS) == 0
    b_stage = b_elems // NS

    mesh = plsc.VectorSubcoreMesh(core_axis_name="c", subcore_axis_name="s")

    @functools.partial(
        pl.kernel,
        mesh=mesh,
        compiler_params=pltpu.CompilerParams(
            needs_layout_passes=False, use_tc_tiling_on_sc=False),
        out_type=jax.ShapeDtypeStruct((B,), jnp.float32),
        scratch_types=[
            pltpu.VMEM((n_per_w,), jnp.int32),
            pltpu.VMEM((n_per_w,), jnp.int32),
            pltpu.VMEM((n_per_w,), jnp.int32),
            pltpu.VMEM((2, 2048), jnp.int32),
            pltpu.VMEM((2, 2048), jnp.int32),
            pltpu.VMEM((2, 2048), jnp.float32),
            pltpu.VMEM((2, 2048), jnp.float32),
            pltpu.VMEM((n_per_w,), jnp.float32),
            pltpu.VMEM_SHARED((b_elems,), jnp.float32),
            pltpu.SemaphoreType.DMA,
            pltpu.SemaphoreType.DMA,
            pltpu.SemaphoreType.DMA,
            pltpu.SemaphoreType.DMA,
            pltpu.SemaphoreType.DMA,
        ],
    )
    def k(layer_hbm, aidx_hbm, bidx_hbm, a_hbm, b_hbm, out_hbm,
          layer_v, aidx_v, bidx_v, idxa_v, idxb_v, arows_v, brows_v, out_v,
          bsh, sem_a0, sem_a1, sem_b0, sem_b1, sem_st):
        sid = lax.axis_index("s")
        wid = sid * NC + lax.axis_index("c")
        base = wid * n_per_w
        sems_a = (sem_a0, sem_a1)
        sems_b = (sem_b0, sem_b1)
        n_chunks = 4
        cg = n_groups // n_chunks
        cn = n_per_w // n_chunks
        assert cg == 8 and cn == 128

        ssl = pl.ds(sid * b_stage, b_stage)
        cp_st = pltpu.async_copy(b_hbm.at[ssl], bsh.at[ssl], sem_st)

        pltpu.sync_copy(layer_hbm.at[pl.ds(base, n_per_w)], layer_v)
        pltpu.sync_copy(aidx_hbm.at[pl.ds(base, n_per_w)], aidx_v)
        pltpu.sync_copy(bidx_hbm.at[pl.ds(base, n_per_w)], bidx_v)

        def build(c):
            p = c % 2

            def idx_body(g, _):
                off = g * _L
                loc = off - c * cn
                sl = pl.ds(off, _L)
                ra = layer_v[sl] * NUM + aidx_v[sl]
                rb = bidx_v[sl]
                jca = ((ra >> 7) << 10) + (ra & 127)
                jcb = ((rb >> 7) << 10) + (rb & 127)
                for d in range(_L):
                    ca = ((d >> 3) * (8 * n_rows_a)) + ((d & 7) << 7)
                    cb = ((d >> 3) * (8 * n_rows_b)) + ((d & 7) << 7)
                    idxa_v[p, pl.ds(d * cn + loc, _L)] = jca + ca
                    idxb_v[p, pl.ds(d * cn + loc, _L)] = jcb + cb
                return 0

            lax.fori_loop(c * cg, (c + 1) * cg, idx_body, 0)

        def fire_a(c):
            p = c % 2
            return pltpu.async_copy(
                a_hbm.at[idxa_v.at[p]], arows_v.at[p], sems_a[p])

        def fire_b(c):
            p = c % 2
            return pltpu.async_copy(
                bsh.at[idxb_v.at[p]], brows_v.at[p], sems_b[p])

        def dot(c):
            p = c % 2

            def dot_body(g, _):
                off = g * _L
                loc = off - c * cn
                acc = jnp.zeros((_L,), jnp.float32)
                for d in range(_L):
                    sl = pl.ds(d * cn + loc, _L)
                    acc = acc + arows_v[p, sl] * brows_v[p, sl]
                out_v[pl.ds(off, _L)] = acc
                return 0

            lax.fori_loop(c * cg, (c + 1) * cg, dot_body, 0)

        build(0)
        cpa0 = fire_a(0)
        build(1)
        cpa1 = fire_a(1)
        cp_st.wait()
        plsc.subcore_barrier()
        cpb0 = fire_b(0)
        cpb1 = fire_b(1)
        cpa0.wait()
        cpb0.wait()
        dot(0)
        build(2)
        cpa2 = fire_a(2)
        cpb2 = fire_b(2)
        cpa1.wait()
        cpb1.wait()
        dot(1)
        build(3)
        cpa3 = fire_a(3)
        cpb3 = fire_b(3)
        cpa2.wait()
        cpb2.wait()
        dot(2)
        cpa3.wait()
        cpb3.wait()
        dot(3)

        pltpu.sync_copy(out_v, out_hbm.at[pl.ds(base, n_per_w)])

    return k


def kernel(layerIdx, aIdx, bIdx, A_table, B_table):
    B = layerIdx.shape[0]
    NUM, D = B_table.shape
    n_rows_a = A_table.shape[0]
    assert n_rows_a % 128 == 0 and D % 8 == 0
    pad_b = (-NUM) % 128
    b_padded = jnp.pad(B_table, ((0, pad_b), (0, 0)))
    a_flat = _flat_view(A_table)
    b_flat = _flat_view(b_padded)
    k = _build(B, D, NUM, n_rows_a, NUM + pad_b)
    return k(layerIdx.astype(jnp.int32), aIdx.astype(jnp.int32),
             bIdx.astype(jnp.int32), a_flat, b_flat)
